# baseline (speedup 1.0000x reference)
import jax
import jax.numpy as jnp
from jax.experimental import pallas as pl
from jax.experimental.pallas import tpu as pltpu

BATCH = 16384
HIDDEN = 256
CHUNK = 2048
NCHUNK = BATCH // CHUNK


def _body(bias_ref, a_hbm, w_ref, o_ref, buf0, buf1, sem0, sem1):
    bufs = (buf0, buf1)
    sems = (sem0, sem1)

    def start(ci):
        return pltpu.make_async_copy(
            a_hbm.at[pl.ds(ci * CHUNK, CHUNK)], bufs[ci % 2], sems[ci % 2])

    start(0).start()
    start(1).start()
    for ci in range(NCHUNK):
        start(ci).wait()
        h = jnp.clip(bufs[ci % 2][...], 0.0, 1.0)
        res = jax.lax.dot_general(
            w_ref[...], h, (((1,), (1,)), ((), ())),
            preferred_element_type=jnp.float32)
        o_ref[pl.ds(ci * CHUNK, CHUNK)] = res[0] + bias_ref[0]
        if ci + 2 < NCHUNK:
            start(ci + 2).start()


def kernel(accumulator, output_weights, output_bias):
    bias = jnp.reshape(output_bias, (1,)).astype(jnp.float32)
    w2d = jnp.reshape(output_weights, (1, HIDDEN))
    out = pl.pallas_call(
        _body,
        in_specs=[
            pl.BlockSpec(memory_space=pltpu.MemorySpace.SMEM),
            pl.BlockSpec(memory_space=pltpu.MemorySpace.HBM),
            pl.BlockSpec((1, HIDDEN), lambda: (0, 0)),
        ],
        out_specs=pl.BlockSpec((BATCH,), lambda: (0,)),
        out_shape=jax.ShapeDtypeStruct((BATCH,), jnp.float32),
        scratch_shapes=[
            pltpu.VMEM((CHUNK, HIDDEN), jnp.float32),
            pltpu.VMEM((CHUNK, HIDDEN), jnp.float32),
            pltpu.SemaphoreType.DMA,
            pltpu.SemaphoreType.DMA,
        ],
    )(bias, accumulator, w2d)
    return out


# speedup vs baseline: 1.0205x; 1.0205x over previous
import jax
import jax.numpy as jnp
from jax.experimental import pallas as pl
from jax.experimental.pallas import tpu as pltpu

BATCH = 16384
HIDDEN = 256
CHUNK = 1024
NCHUNK = BATCH // CHUNK
NBUF = 4


def _body(bias_ref, a_hbm, w_ref, o_ref, *rest):
    bufs, sems = rest[:NBUF], rest[NBUF:]

    def start(ci):
        return pltpu.make_async_copy(
            a_hbm.at[pl.ds(ci * CHUNK, CHUNK)], bufs[ci % NBUF], sems[ci % NBUF])

    for ci in range(NBUF - 1):
        start(ci).start()
    for ci in range(NCHUNK):
        start(ci).wait()
        if ci + NBUF - 1 < NCHUNK:
            start(ci + NBUF - 1).start()
        h = jnp.clip(bufs[ci % NBUF][...], 0.0, 1.0)
        res = jax.lax.dot_general(
            w_ref[...], h, (((1,), (1,)), ((), ())),
            preferred_element_type=jnp.float32)
        o_ref[pl.ds(ci * CHUNK, CHUNK)] = res[0] + bias_ref[0]


def kernel(accumulator, output_weights, output_bias):
    bias = jnp.reshape(output_bias, (1,)).astype(jnp.float32)
    w2d = jnp.reshape(output_weights, (1, HIDDEN))
    out = pl.pallas_call(
        _body,
        in_specs=[
            pl.BlockSpec(memory_space=pltpu.MemorySpace.SMEM),
            pl.BlockSpec(memory_space=pltpu.MemorySpace.HBM),
            pl.BlockSpec((1, HIDDEN), lambda: (0, 0)),
        ],
        out_specs=pl.BlockSpec((BATCH,), lambda: (0,)),
        out_shape=jax.ShapeDtypeStruct((BATCH,), jnp.float32),
        scratch_shapes=(
            [pltpu.VMEM((CHUNK, HIDDEN), jnp.float32) for _ in range(NBUF)]
            + [pltpu.SemaphoreType.DMA for _ in range(NBUF)]
        ),
    )(bias, accumulator, w2d)
    return out


# speedup vs baseline: 1.0672x; 1.0457x over previous
import jax
import jax.numpy as jnp
from jax.experimental import pallas as pl
from jax.experimental.pallas import tpu as pltpu

BATCH = 16384
HIDDEN = 256
BLOCK_ROWS = 4096
HALF = BATCH // 2
HALF_BLOCKS = HALF // BLOCK_ROWS


def _body(bias_ref, a_ref, b_ref, w_ref, o_ref):
    w = w_ref[...]
    dn = (((1,), (1,)), ((), ()))
    h1 = jnp.clip(a_ref[...], 0.0, 1.0)
    r1 = jax.lax.dot_general(w, h1, dn, preferred_element_type=jnp.float32)
    o_ref[0, :] = r1[0] + bias_ref[0]
    h2 = jnp.clip(b_ref[...], 0.0, 1.0)
    r2 = jax.lax.dot_general(w, h2, dn, preferred_element_type=jnp.float32)
    o_ref[1, :] = r2[0] + bias_ref[0]


def kernel(accumulator, output_weights, output_bias):
    bias = jnp.reshape(output_bias, (1,)).astype(jnp.float32)
    w2d = jnp.reshape(output_weights, (1, HIDDEN))
    grid = (HALF_BLOCKS,)
    out = pl.pallas_call(
        _body,
        grid=grid,
        in_specs=[
            pl.BlockSpec(memory_space=pltpu.MemorySpace.SMEM),
            pl.BlockSpec((BLOCK_ROWS, HIDDEN), lambda i: (i, 0)),
            pl.BlockSpec((BLOCK_ROWS, HIDDEN), lambda i: (i + HALF_BLOCKS, 0)),
            pl.BlockSpec((1, HIDDEN), lambda i: (0, 0)),
        ],
        out_specs=pl.BlockSpec((2, BLOCK_ROWS), lambda i: (0, i)),
        out_shape=jax.ShapeDtypeStruct((2, HALF), jnp.float32),
    )(bias, accumulator, accumulator, w2d)
    return jnp.reshape(out, (BATCH,))


# speedup vs baseline: 1.2631x; 1.1835x over previous
import jax
import jax.numpy as jnp
from jax.experimental import pallas as pl
from jax.experimental.pallas import tpu as pltpu

BATCH = 16384
HIDDEN = 256
BLOCK_ROWS = 8192


def _body(bias_ref, a_ref, w_ref, o_ref):
    h = jnp.clip(a_ref[...], 0.0, 1.0)
    res = jax.lax.dot_general(
        w_ref[...], h, (((1,), (1,)), ((), ())),
        preferred_element_type=jnp.float32)
    o_ref[...] = res[0] + bias_ref[0]


def kernel(accumulator, output_weights, output_bias):
    bias = jnp.reshape(output_bias, (1,)).astype(jnp.float32)
    w2d = jnp.reshape(output_weights, (1, HIDDEN))
    grid = (BATCH // BLOCK_ROWS,)
    out = pl.pallas_call(
        _body,
        grid=grid,
        in_specs=[
            pl.BlockSpec(memory_space=pltpu.MemorySpace.SMEM),
            pl.BlockSpec((BLOCK_ROWS, HIDDEN), lambda i: (i, 0)),
            pl.BlockSpec((1, HIDDEN), lambda i: (0, 0)),
        ],
        out_specs=pl.BlockSpec((BLOCK_ROWS,), lambda i: (i,)),
        out_shape=jax.ShapeDtypeStruct((BATCH,), jnp.float32),
    )(bias, accumulator, w2d)
    return out
